# baseline (speedup 1.0000x reference)
import functools

import jax
import jax.numpy as jnp
from jax import lax
from jax.experimental import pallas as pl
from jax.experimental.pallas import tpu as pltpu
from jax.experimental.pallas import tpu_sc as plsc

ED = 64
CTX = 20
K = 20
NC = 2
NS = 16
NWK = NC * NS
CHUNK = 16
LANES = 16
PIECES = ((0, 128), (128, 128), (256, 64))
TP = 640
TPW = 3200

_params = pltpu.CompilerParams(
    needs_layout_passes=False, use_tc_tiling_on_sc=False)


def _tr_body(zero_tail, wp_hbm, wpt_hbm, slab_a, slab_b, otile, sem_a, sem_b):
    wid = lax.axis_index("s") * NC + lax.axis_index("c")
    nw = wp_hbm.shape[1]
    percore = -(-nw // NWK)
    cov = (nw - TPW) // 8 * 8 + TPW
    c0 = jnp.minimum((wid * percore) // 8 * 8, (nw - TPW) // 8 * 8)
    npiece = TPW // TP
    lanes = jnp.arange(LANES, dtype=jnp.int32)

    def issue(p, slab, sem):
        off = pl.multiple_of(c0 + p * TP, 8)
        pltpu.async_copy(wp_hbm.at[:, pl.ds(off, TP)], slab, sem)

    def drain(slab, sem):
        off = pl.multiple_of(c0, 8)
        pltpu.make_async_copy(wp_hbm.at[:, pl.ds(off, TP)], slab, sem).wait()

    def transpose_piece(p, slab):
        def w_body(w, carry):
            for v in range(4):
                rows = v * LANES + lanes
                cols = jnp.full((LANES,), w, jnp.int32)
                vals = plsc.load_gather(slab, [rows, cols])
                otile[w, pl.ds(v * LANES, LANES)] = vals
            return carry
        lax.fori_loop(0, TP, w_body, 0)
        pltpu.sync_copy(otile, wpt_hbm.at[pl.ds(c0 + p * TP, TP)])

    issue(0, slab_a, sem_a)
    for p in range(npiece):
        slab, sem = (slab_a, sem_a) if p % 2 == 0 else (slab_b, sem_b)
        nslab, nsem = (slab_b, sem_b) if p % 2 == 0 else (slab_a, sem_a)
        drain(slab, sem)
        if p + 1 < npiece:
            issue(p + 1, nslab, nsem)
        transpose_piece(p, slab)

    if zero_tail and cov < nw:
        @pl.when(wid == NWK - 1)
        def _tail():
            zeros = jnp.zeros((LANES,), jnp.float32)
            for v in range(4):
                otile[0, pl.ds(v * LANES, LANES)] = zeros
            for r in range(nw - cov):
                pltpu.sync_copy(otile.at[pl.ds(0, 1)],
                                wpt_hbm.at[pl.ds(cov + r, 1)])


def _make_transpose(nw, zero_tail):
    mesh = plsc.VectorSubcoreMesh(core_axis_name="c", subcore_axis_name="s")
    return pl.kernel(
        functools.partial(_tr_body, zero_tail),
        out_type=jax.ShapeDtypeStruct((nw, ED), jnp.float32),
        mesh=mesh,
        scratch_types=[
            pltpu.VMEM((ED, TP), jnp.float32),
            pltpu.VMEM((ED, TP), jnp.float32),
            pltpu.VMEM((TP, ED), jnp.float32),
            pltpu.SemaphoreType.DMA,
            pltpu.SemaphoreType.DMA,
        ],
        compiler_params=_params,
    )


def _dm_body(ctx_hbm, doc_hbm, tn_hbm, d_hbm, w_hbm, wpt_hbm, out_hbm,
             ctx_idx, tn_idx, doc_idx,
             gc_a, gt_a, gd_a, gc_b, gt_b, gd_b, out_v, sem_a, sem_b):
    wid = lax.axis_index("s") * NC + lax.axis_index("c")
    nb = doc_hbm.shape[0] // NWK
    b0 = wid * nb

    pltpu.sync_copy(ctx_hbm.at[pl.ds(b0 * CTX, nb * CTX)], ctx_idx)
    pltpu.sync_copy(tn_hbm.at[pl.ds(b0 * K, nb * K)], tn_idx)
    pltpu.sync_copy(doc_hbm.at[pl.ds(b0, nb)], doc_idx)

    lanes = jnp.arange(LANES, dtype=jnp.int32)
    nchunks = nb // CHUNK

    def issue(c, gc, gt, gd, sem):
        ib = c * (CHUNK * CTX)
        for off, n in PIECES:
            pltpu.async_copy(w_hbm.at[ctx_idx.at[pl.ds(ib + off, n)]],
                             gc.at[pl.ds(off, n)], sem)
            pltpu.async_copy(wpt_hbm.at[tn_idx.at[pl.ds(ib + off, n)]],
                             gt.at[pl.ds(off, n)], sem)
        pltpu.async_copy(d_hbm.at[doc_idx.at[pl.ds(c * CHUNK, CHUNK)]],
                         gd, sem)

    def drain(gc, gt, gd, sem):
        for off, n in PIECES:
            pltpu.make_async_copy(w_hbm.at[ctx_idx.at[pl.ds(0, n)]],
                                  gc.at[pl.ds(off, n)], sem).wait()
            pltpu.make_async_copy(wpt_hbm.at[tn_idx.at[pl.ds(0, n)]],
                                  gt.at[pl.ds(off, n)], sem).wait()
        pltpu.make_async_copy(d_hbm.at[doc_idx.at[pl.ds(0, CHUNK)]],
                              gd, sem).wait()

    def compute(c, gc, gt, gd):
        o0 = c * (CHUNK * K)

        def b_body(i, carry2):
            r0 = i * CTX
            acc = [gd[i, pl.ds(v * LANES, LANES)] for v in range(4)]
            for j in range(CTX):
                for v in range(4):
                    acc[v] = acc[v] + gc[r0 + j, pl.ds(v * LANES, LANES)]
            ov0 = jnp.zeros((LANES,), jnp.float32)
            ov1 = jnp.zeros((LANES,), jnp.float32)
            for k in range(K):
                p = acc[0] * gt[r0 + k, pl.ds(0, LANES)]
                for v in range(1, 4):
                    p = p + acc[v] * gt[r0 + k, pl.ds(v * LANES, LANES)]
                s = jnp.sum(p)
                sv = jnp.full((LANES,), s, jnp.float32)
                if k < LANES:
                    ov0 = jnp.where(lanes == k, sv, ov0)
                else:
                    ov1 = jnp.where(lanes == (k - LANES), sv, ov1)
            out_v[pl.ds(o0 + i * K, LANES)] = ov0
            out_v[pl.ds(o0 + i * K + LANES, LANES)] = ov1
            return carry2

        lax.fori_loop(0, CHUNK, b_body, 0)

    issue(0, gc_a, gt_a, gd_a, sem_a)
    nsteps = nchunks // 2

    def step(t, carry):
        c = 2 * t
        drain(gc_a, gt_a, gd_a, sem_a)
        issue(c + 1, gc_b, gt_b, gd_b, sem_b)
        compute(c, gc_a, gt_a, gd_a)
        drain(gc_b, gt_b, gd_b, sem_b)

        @pl.when(t < nsteps - 1)
        def _prefetch():
            issue(c + 2, gc_a, gt_a, gd_a, sem_a)

        compute(c + 1, gc_b, gt_b, gd_b)
        return carry

    lax.fori_loop(0, nsteps, step, 0)
    pltpu.sync_copy(out_v.at[pl.ds(0, nb * K)],
                    out_hbm.at[pl.ds(b0 * K, nb * K)])


def _make_kernel(B):
    nb = B // NWK
    mesh = plsc.VectorSubcoreMesh(core_axis_name="c", subcore_axis_name="s")
    gather_bufs = [
        pltpu.VMEM((CHUNK * CTX, ED), jnp.float32),
        pltpu.VMEM((CHUNK * K, ED), jnp.float32),
        pltpu.VMEM((CHUNK, ED), jnp.float32),
    ]
    return pl.kernel(
        _dm_body,
        out_type=jax.ShapeDtypeStruct((B * K,), jnp.float32),
        mesh=mesh,
        scratch_types=[
            pltpu.VMEM((nb * CTX,), jnp.int32),
            pltpu.VMEM((nb * K,), jnp.int32),
            pltpu.VMEM((nb,), jnp.int32),
            *gather_bufs,
            *gather_bufs,
            pltpu.VMEM((nb * K + LANES,), jnp.float32),
            pltpu.SemaphoreType.DMA,
            pltpu.SemaphoreType.DMA,
        ],
        compiler_params=pltpu.CompilerParams(
            needs_layout_passes=False, use_tc_tiling_on_sc=False),
    )


def kernel(ctx_ids, doc_ids, target_and_noise_ids, D, W, Wp):
    B = ctx_ids.shape[0]
    WpT = Wp.T
    Wl = _make_transpose(W.shape[0], True)(W.T)
    Dl = _make_transpose(D.shape[0], False)(D.T)
    out = _make_kernel(B)(
        ctx_ids.reshape(-1), doc_ids, target_and_noise_ids.reshape(-1),
        Dl, Wl, WpT)
    return out.reshape(B, K)


# speedup vs baseline: 2.6627x; 2.6627x over previous
import functools

import jax
import jax.numpy as jnp
from jax import lax
from jax.experimental import pallas as pl
from jax.experimental.pallas import tpu as pltpu
from jax.experimental.pallas import tpu_sc as plsc

ED = 64
CTX = 20
K = 20
NC = 2
NS = 16
NWK = NC * NS
CHUNK = 16
LANES = 16
PIECES = ((0, 128), (128, 128), (256, 64))
TP = 640
TPW = 3200

_params = pltpu.CompilerParams(
    needs_layout_passes=False, use_tc_tiling_on_sc=False)


TP2 = 504
NP2 = 7
TPW2 = TP2 * NP2


def _tr_body(src_hbm, dst_hbm,
             slab_a, slab_b, otile_a, otile_b, sem_a, sem_b, sem_o):
    wid = lax.axis_index("s") * NC + lax.axis_index("c")
    lanes = jnp.arange(LANES, dtype=jnp.int32)
    nw = src_hbm.shape[1]
    percore = -(-nw // NWK)
    c0 = jnp.minimum((wid * percore) // 8 * 8, (nw - TPW2) // 8 * 8)

    def start(p, slab, sem):
        off = pl.multiple_of(c0 + p * TP2, 8)
        pltpu.async_copy(src_hbm.at[:, pl.ds(off, TP2)], slab, sem)

    def wait_in(slab, sem):
        off = pl.multiple_of(c0, 8)
        pltpu.make_async_copy(src_hbm.at[:, pl.ds(off, TP2)], slab,
                              sem).wait()

    def wait_out(otile):
        pltpu.make_async_copy(otile, dst_hbm.at[pl.ds(c0, TP2)],
                              sem_o).wait()

    def transpose_piece(slab, otile):
        def w_body(w, carry):
            for v in range(4):
                rows = v * LANES + lanes
                cols = jnp.full((LANES,), w, jnp.int32)
                vals = plsc.load_gather(slab, [rows, cols])
                otile[w, pl.ds(v * LANES, LANES)] = vals
            return carry
        lax.fori_loop(0, TP2, w_body, 0)

    bufs = ((slab_a, sem_a, otile_a), (slab_b, sem_b, otile_b))
    start(0, slab_a, sem_a)
    for p in range(NP2):
        slab, sem, otile = bufs[p % 2]
        wait_in(slab, sem)
        if p + 1 < NP2:
            start(p + 1, *bufs[(p + 1) % 2][:2])
        if p >= 2:
            wait_out(otile)
        transpose_piece(slab, otile)
        pltpu.async_copy(otile, dst_hbm.at[pl.ds(c0 + p * TP2, TP2)], sem_o)
    wait_out(otile_a if (NP2 - 2) % 2 == 0 else otile_b)
    wait_out(otile_a if (NP2 - 1) % 2 == 0 else otile_b)


def _make_transpose(nw):
    mesh = plsc.VectorSubcoreMesh(core_axis_name="c", subcore_axis_name="s")
    return pl.kernel(
        _tr_body,
        out_type=jax.ShapeDtypeStruct((nw, ED), jnp.float32),
        mesh=mesh,
        scratch_types=[
            pltpu.VMEM((ED, TP2), jnp.float32),
            pltpu.VMEM((ED, TP2), jnp.float32),
            pltpu.VMEM((TP2, ED), jnp.float32),
            pltpu.VMEM((TP2, ED), jnp.float32),
            pltpu.SemaphoreType.DMA,
            pltpu.SemaphoreType.DMA,
            pltpu.SemaphoreType.DMA,
        ],
        compiler_params=_params,
    )


def _dm_body(ctx_hbm, doc_hbm, tn_hbm, d_hbm, w_hbm, wpt_hbm, out_hbm,
             ctx_idx, tn_idx, doc_idx,
             gc_a, gt_a, gd_a, gc_b, gt_b, gd_b, out_v, sem_a, sem_b):
    wid = lax.axis_index("s") * NC + lax.axis_index("c")
    nb = doc_hbm.shape[0] // NWK
    b0 = wid * nb

    h1 = pltpu.async_copy(ctx_hbm.at[pl.ds(b0 * CTX, nb * CTX)], ctx_idx,
                          sem_a)
    h2 = pltpu.async_copy(tn_hbm.at[pl.ds(b0 * K, nb * K)], tn_idx, sem_a)
    h3 = pltpu.async_copy(doc_hbm.at[pl.ds(b0, nb)], doc_idx, sem_a)
    h1.wait()
    h2.wait()
    h3.wait()

    lanes = jnp.arange(LANES, dtype=jnp.int32)
    nchunks = nb // CHUNK

    def issue(c, gc, gt, gd, sem):
        ib = c * (CHUNK * CTX)
        for off, n in PIECES:
            pltpu.async_copy(w_hbm.at[ctx_idx.at[pl.ds(ib + off, n)]],
                             gc.at[pl.ds(off, n)], sem)
            pltpu.async_copy(wpt_hbm.at[tn_idx.at[pl.ds(ib + off, n)]],
                             gt.at[pl.ds(off, n)], sem)
        pltpu.async_copy(d_hbm.at[doc_idx.at[pl.ds(c * CHUNK, CHUNK)]],
                         gd, sem)

    def drain(gc, gt, gd, sem):
        for off, n in PIECES:
            pltpu.make_async_copy(w_hbm.at[ctx_idx.at[pl.ds(0, n)]],
                                  gc.at[pl.ds(off, n)], sem).wait()
            pltpu.make_async_copy(wpt_hbm.at[tn_idx.at[pl.ds(0, n)]],
                                  gt.at[pl.ds(off, n)], sem).wait()
        pltpu.make_async_copy(d_hbm.at[doc_idx.at[pl.ds(0, CHUNK)]],
                              gd, sem).wait()

    def compute(c, gc, gt, gd):
        o0 = c * (CHUNK * K)

        def b_body(i, carry2):
            r0 = i * CTX
            acc = [gd[i, pl.ds(v * LANES, LANES)] for v in range(4)]
            for j in range(CTX):
                for v in range(4):
                    acc[v] = acc[v] + gc[r0 + j, pl.ds(v * LANES, LANES)]
            ov0 = jnp.zeros((LANES,), jnp.float32)
            ov1 = jnp.zeros((LANES,), jnp.float32)
            for k in range(K):
                p = acc[0] * gt[r0 + k, pl.ds(0, LANES)]
                for v in range(1, 4):
                    p = p + acc[v] * gt[r0 + k, pl.ds(v * LANES, LANES)]
                s = jnp.sum(p)
                sv = jnp.full((LANES,), s, jnp.float32)
                if k < LANES:
                    ov0 = jnp.where(lanes == k, sv, ov0)
                else:
                    ov1 = jnp.where(lanes == (k - LANES), sv, ov1)
            out_v[pl.ds(o0 + i * K, LANES)] = ov0
            out_v[pl.ds(o0 + i * K + LANES, LANES)] = ov1
            return carry2

        lax.fori_loop(0, CHUNK, b_body, 0)

    issue(0, gc_a, gt_a, gd_a, sem_a)
    nsteps = nchunks // 2

    def step(t, carry):
        c = 2 * t
        drain(gc_a, gt_a, gd_a, sem_a)
        issue(c + 1, gc_b, gt_b, gd_b, sem_b)
        compute(c, gc_a, gt_a, gd_a)
        drain(gc_b, gt_b, gd_b, sem_b)

        @pl.when(t < nsteps - 1)
        def _prefetch():
            issue(c + 2, gc_a, gt_a, gd_a, sem_a)

        compute(c + 1, gc_b, gt_b, gd_b)
        return carry

    lax.fori_loop(0, nsteps, step, 0)
    pltpu.sync_copy(out_v.at[pl.ds(0, nb * K)],
                    out_hbm.at[pl.ds(b0 * K, nb * K)])


def _make_kernel(B):
    nb = B // NWK
    mesh = plsc.VectorSubcoreMesh(core_axis_name="c", subcore_axis_name="s")
    gather_bufs = [
        pltpu.VMEM((CHUNK * CTX, ED), jnp.float32),
        pltpu.VMEM((CHUNK * K, ED), jnp.float32),
        pltpu.VMEM((CHUNK, ED), jnp.float32),
    ]
    return pl.kernel(
        _dm_body,
        out_type=jax.ShapeDtypeStruct((B * K,), jnp.float32),
        mesh=mesh,
        scratch_types=[
            pltpu.VMEM((nb * CTX,), jnp.int32),
            pltpu.VMEM((nb * K,), jnp.int32),
            pltpu.VMEM((nb,), jnp.int32),
            *gather_bufs,
            *gather_bufs,
            pltpu.VMEM((nb * K + LANES,), jnp.float32),
            pltpu.SemaphoreType.DMA,
            pltpu.SemaphoreType.DMA,
        ],
        compiler_params=pltpu.CompilerParams(
            needs_layout_passes=False, use_tc_tiling_on_sc=False),
    )


def kernel(ctx_ids, doc_ids, target_and_noise_ids, D, W, Wp):
    B = ctx_ids.shape[0]
    WpT = _make_transpose(Wp.shape[1])(Wp)
    out = _make_kernel(B)(
        ctx_ids.reshape(-1), doc_ids, target_and_noise_ids.reshape(-1),
        D, W, WpT)
    return out.reshape(B, K)
